# Initial kernel scaffold; baseline (speedup 1.0000x reference)
import functools

import jax
import jax.numpy as jnp
from jax import lax
from jax.experimental import pallas as pl
from jax.experimental.pallas import tpu as pltpu
from jax.experimental.pallas import tpu_sc as plsc

_B = 16384
_H = 50
_D = 32
_NC = 2
_NS = 16
_NW = _NC * _NS
_BPW = _B // _NW
_CB = 32
_NCHUNK = _BPW // _CB
_G = 100
_GPC = _CB * _H // _G


def _bow_body(idx_hbm, table_hbm, bias_hbm, out_hbm,
              idx_v, rows_v, out_v, bias_v, sem):
    wid = lax.axis_index("s") * _NC + lax.axis_index("c")
    base_row = wid * _BPW

    pltpu.sync_copy(bias_hbm, bias_v)

    def chunk_body(c, carry):
        row0 = base_row + c * _CB
        irow0 = row0 * _H // _G
        pltpu.sync_copy(idx_hbm.at[pl.ds(irow0, _GPC)], idx_v)
        copies = []
        for r in range(_GPC):
            copies.append(pltpu.async_copy(
                table_hbm.at[idx_v.at[r]],
                rows_v.at[pl.ds(r * _G, _G)], sem))
        for cp in copies:
            cp.wait()

        bias0 = bias_v[pl.ds(0, 16)]
        bias1 = bias_v[pl.ds(16, 16)]

        def acc_body(b, carry2):
            r0 = b * _H
            a0 = bias0
            a1 = bias1
            b0 = rows_v[r0, pl.ds(0, 16)]
            b1 = rows_v[r0, pl.ds(16, 16)]
            for h in range(1, _H, 2):
                a0 = a0 + rows_v[r0 + h, pl.ds(0, 16)]
                a1 = a1 + rows_v[r0 + h, pl.ds(16, 16)]
                if h + 1 < _H:
                    b0 = b0 + rows_v[r0 + h + 1, pl.ds(0, 16)]
                    b1 = b1 + rows_v[r0 + h + 1, pl.ds(16, 16)]
            out_v[b, pl.ds(0, 16)] = a0 + b0
            out_v[b, pl.ds(16, 16)] = a1 + b1
            return carry2

        lax.fori_loop(0, _CB, acc_body, 0)
        pltpu.sync_copy(out_v, out_hbm.at[pl.ds(row0, _CB)])
        return carry

    lax.fori_loop(0, _NCHUNK, chunk_body, 0)


@jax.jit
def kernel(inputs, table, bias):
    idx2d = inputs.astype(jnp.int32).reshape(_B * _H // _G, _G)
    mesh = plsc.VectorSubcoreMesh(
        core_axis_name="c", subcore_axis_name="s",
        num_cores=_NC, num_subcores=_NS)
    k = functools.partial(
        pl.kernel,
        out_type=jax.ShapeDtypeStruct((_B, _D), jnp.float32),
        mesh=mesh,
        scratch_types=[
            pltpu.VMEM((_GPC, _G), jnp.int32),
            pltpu.VMEM((_CB * _H, _D), jnp.float32),
            pltpu.VMEM((_CB, _D), jnp.float32),
            pltpu.VMEM((_D,), jnp.float32),
            pltpu.SemaphoreType.DMA,
        ],
    )(_bow_body)
    return k(idx2d, table, bias)


# speedup vs baseline: 2.7464x; 2.7464x over previous
import functools

import jax
import jax.numpy as jnp
from jax import lax
from jax.experimental import pallas as pl
from jax.experimental.pallas import tpu as pltpu
from jax.experimental.pallas import tpu_sc as plsc

_B = 16384
_H = 50
_D = 32
_NC = 2
_NS = 16
_NW = _NC * _NS
_BPW = _B // _NW
_CB = 32
_NCHUNK = _BPW // _CB
_G = 100
_GPC = _CB * _H // _G


def _bow_body(idx_hbm, table_hbm, bias_hbm, out_hbm,
              idx_v, rows_v, out_v, bias_v, sem):
    wid = lax.axis_index("s") * _NC + lax.axis_index("c")
    base_row = wid * _BPW

    pltpu.sync_copy(bias_hbm, bias_v)

    def chunk_body(c, carry):
        row0 = pl.multiple_of(base_row + c * _CB, _CB)
        irow0 = pl.multiple_of(row0 * _H // _G, _GPC)
        pltpu.sync_copy(idx_hbm.at[pl.ds(irow0, _GPC)], idx_v)
        copies = []
        for r in range(_GPC):
            copies.append(pltpu.async_copy(
                table_hbm.at[idx_v.at[r]],
                rows_v.at[pl.ds(r * _G, _G)], sem))
        for cp in copies:
            cp.wait()

        bias0 = bias_v[pl.ds(0, 16)]
        bias1 = bias_v[pl.ds(16, 16)]

        def acc_body(b, carry2):
            r0 = b * _H
            a0 = bias0
            a1 = bias1
            b0 = rows_v[r0, pl.ds(0, 16)]
            b1 = rows_v[r0, pl.ds(16, 16)]
            for h in range(1, _H, 2):
                a0 = a0 + rows_v[r0 + h, pl.ds(0, 16)]
                a1 = a1 + rows_v[r0 + h, pl.ds(16, 16)]
                if h + 1 < _H:
                    b0 = b0 + rows_v[r0 + h + 1, pl.ds(0, 16)]
                    b1 = b1 + rows_v[r0 + h + 1, pl.ds(16, 16)]
            out_v[b, pl.ds(0, 16)] = a0 + b0
            out_v[b, pl.ds(16, 16)] = a1 + b1
            return carry2

        lax.fori_loop(0, _CB, acc_body, 0)
        pltpu.sync_copy(out_v, out_hbm.at[pl.ds(row0, _CB)])
        return carry

    lax.fori_loop(0, _NCHUNK, chunk_body, 0)


@jax.jit
def kernel(inputs, table, bias):
    idx2d = inputs.astype(jnp.int32).reshape(_B * _H // _G, _G)
    mesh = plsc.VectorSubcoreMesh(
        core_axis_name="c", subcore_axis_name="s",
        num_cores=_NC, num_subcores=_NS)
    k = functools.partial(
        pl.kernel,
        out_type=jax.ShapeDtypeStruct((_B, _D), jnp.float32),
        mesh=mesh,
        scratch_types=[
            pltpu.VMEM((_GPC, _G), jnp.int32),
            pltpu.VMEM((_CB * _H, _D), jnp.float32),
            pltpu.VMEM((_CB, _D), jnp.float32),
            pltpu.VMEM((_D,), jnp.float32),
            pltpu.SemaphoreType.DMA,
        ],
        compiler_params=pltpu.CompilerParams(use_tc_tiling_on_sc=False),
    )(_bow_body)
    return k(idx2d, table, bias)
